# baseline (speedup 1.0000x reference)
import functools

import jax
import jax.numpy as jnp
from jax import lax
from jax.experimental import pallas as pl
from jax.experimental.pallas import tpu as pltpu
from jax.experimental.pallas import tpu_sc as plsc

R = 8
L = 16


def _body(B, S, D, x_hbm, t_hbm, o_hbm, tbuf, xbuf):
    info = plsc.get_sparse_core_info()
    nw = info.num_cores * info.num_subcores
    wid = lax.axis_index("s") * info.num_cores + lax.axis_index("c")
    rows_per_w = S // nw
    base = wid * rows_per_w
    cw = R * D
    nvec = cw // L

    def chunk_body(c, _):
        s0 = base + c * R
        pltpu.sync_copy(t_hbm.at[pl.ds(s0 * D, cw)], tbuf)
        for b in range(B):
            pltpu.sync_copy(x_hbm.at[pl.ds((b * S + s0) * D, cw)], xbuf.at[b])

        def vec_body(i, _):
            tv = tbuf[pl.ds(i * L, L)]
            for b in range(B):
                xbuf[b, pl.ds(i * L, L)] = xbuf[b, pl.ds(i * L, L)] + tv
            return ()

        lax.fori_loop(0, nvec, vec_body, ())
        for b in range(B):
            pltpu.sync_copy(xbuf.at[b], o_hbm.at[pl.ds((b * S + s0) * D, cw)])
        return ()

    lax.fori_loop(0, rows_per_w // R, chunk_body, ())


def kernel(input_embeddings, emb_table):
    B, S, D = input_embeddings.shape
    mesh = plsc.VectorSubcoreMesh(core_axis_name="c", subcore_axis_name="s")
    k = pl.kernel(
        functools.partial(_body, B, S, D),
        out_type=jax.ShapeDtypeStruct((B * S * D,), input_embeddings.dtype),
        mesh=mesh,
        scratch_types=[
            pltpu.VMEM((R * D,), jnp.float32),
            pltpu.VMEM((B, R * D), jnp.float32),
        ],
    )
    out = k(input_embeddings.reshape(-1), emb_table[:S].reshape(-1))
    return out.reshape(B, S, D)


# speedup vs baseline: 1.2514x; 1.2514x over previous
import functools

import jax
import jax.numpy as jnp
from jax import lax
from jax.experimental import pallas as pl
from jax.experimental.pallas import tpu as pltpu
from jax.experimental.pallas import tpu_sc as plsc

R = 8
L = 16


def _body(B, S, D, x_hbm, t_hbm, o_hbm, tb0, tb1, xb0, xb1, is0, is1, os0, os1):
    info = plsc.get_sparse_core_info()
    nw = info.num_cores * info.num_subcores
    wid = lax.axis_index("s") * info.num_cores + lax.axis_index("c")
    rows_per_w = S // nw
    base = wid * rows_per_w
    cw = R * D
    nvec = cw // L
    nchunks = rows_per_w // R
    npairs = nchunks // 2

    def start_in(c, tb, xb, isem):
        s0 = base + c * R
        pltpu.async_copy(t_hbm.at[pl.ds(s0 * D, cw)], tb, isem)
        for b in range(B):
            pltpu.async_copy(x_hbm.at[pl.ds((b * S + s0) * D, cw)], xb.at[b], isem)

    def drain_in(tb, xb, isem):
        pltpu.make_async_copy(t_hbm.at[pl.ds(0, cw)], tb, isem).wait()
        for b in range(B):
            pltpu.make_async_copy(x_hbm.at[pl.ds(0, cw)], xb.at[b], isem).wait()

    def start_out(c, xb, osem):
        s0 = base + c * R
        for b in range(B):
            pltpu.async_copy(xb.at[b], o_hbm.at[pl.ds((b * S + s0) * D, cw)], osem)

    def drain_out(xb, osem):
        for b in range(B):
            pltpu.make_async_copy(xb.at[b], o_hbm.at[pl.ds(0, cw)], osem).wait()

    def compute(tb, xb):
        def vec_body(i, _):
            tv = tb[pl.ds(i * L, L)]
            for b in range(B):
                xb[b, pl.ds(i * L, L)] = xb[b, pl.ds(i * L, L)] + tv
            return ()

        lax.fori_loop(0, nvec, vec_body, (), unroll=8)

    start_in(0, tb0, xb0, is0)

    def pair_body(p, _):
        c = 2 * p

        @pl.when(p > 0)
        def _():
            drain_out(xb1, os1)

        start_in(c + 1, tb1, xb1, is1)
        drain_in(tb0, xb0, is0)
        compute(tb0, xb0)
        start_out(c, xb0, os0)

        drain_in(tb1, xb1, is1)

        @pl.when(p < npairs - 1)
        def _():
            drain_out(xb0, os0)
            start_in(c + 2, tb0, xb0, is0)

        compute(tb1, xb1)
        start_out(c + 1, xb1, os1)
        return ()

    lax.fori_loop(0, npairs, pair_body, ())
    drain_out(xb0, os0)
    drain_out(xb1, os1)


def kernel(input_embeddings, emb_table):
    B, S, D = input_embeddings.shape
    mesh = plsc.VectorSubcoreMesh(core_axis_name="c", subcore_axis_name="s")
    k = pl.kernel(
        functools.partial(_body, B, S, D),
        out_type=jax.ShapeDtypeStruct((B * S * D,), input_embeddings.dtype),
        mesh=mesh,
        scratch_types=[
            pltpu.VMEM((R * D,), jnp.float32),
            pltpu.VMEM((R * D,), jnp.float32),
            pltpu.VMEM((B, R * D), jnp.float32),
            pltpu.VMEM((B, R * D), jnp.float32),
            pltpu.SemaphoreType.DMA,
            pltpu.SemaphoreType.DMA,
            pltpu.SemaphoreType.DMA,
            pltpu.SemaphoreType.DMA,
        ],
    )
    out = k(input_embeddings.reshape(-1), emb_table[:S].reshape(-1))
    return out.reshape(B, S, D)


# speedup vs baseline: 1.4236x; 1.1376x over previous
import functools

import jax
import jax.numpy as jnp
from jax import lax
from jax.experimental import pallas as pl
from jax.experimental.pallas import tpu as pltpu
from jax.experimental.pallas import tpu_sc as plsc

R = 8
L = 16


def _body(B, S, D, x_hbm, t_hbm, o_hbm, tb0, tb1, xb0, xb1, is0, is1, os0, os1):
    info = plsc.get_sparse_core_info()
    nw = info.num_cores * info.num_subcores
    wid = lax.axis_index("s") * info.num_cores + lax.axis_index("c")
    rows_per_w = S // nw
    base = wid * rows_per_w
    cw = R * D
    nvec = cw // L
    nchunks = rows_per_w // R
    npairs = nchunks // 2

    def start_in(c, tb, xb, isem):
        s0 = base + c * R
        pltpu.async_copy(t_hbm.at[pl.ds(s0 * D, cw)], tb, isem)
        for b in range(B):
            pltpu.async_copy(x_hbm.at[pl.ds((b * S + s0) * D, cw)], xb.at[b], isem)

    def drain_in(tb, xb, isem):
        pltpu.make_async_copy(t_hbm.at[pl.ds(0, cw)], tb, isem).wait()
        for b in range(B):
            pltpu.make_async_copy(x_hbm.at[pl.ds(0, cw)], xb.at[b], isem).wait()

    def start_out(c, xb, osem):
        s0 = base + c * R
        for b in range(B):
            pltpu.async_copy(xb.at[b], o_hbm.at[pl.ds((b * S + s0) * D, cw)], osem)

    def drain_out(xb, osem):
        for b in range(B):
            pltpu.make_async_copy(xb.at[b], o_hbm.at[pl.ds(0, cw)], osem).wait()

    def compute(tb, xb):
        @plsc.parallel_loop(0, cw, L, unroll=8)
        def _(i):
            tv = tb[pl.ds(i, L)]
            for b in range(B):
                xb[b, pl.ds(i, L)] = xb[b, pl.ds(i, L)] + tv

    start_in(0, tb0, xb0, is0)

    def pair_body(p, _):
        c = 2 * p

        @pl.when(p > 0)
        def _():
            drain_out(xb1, os1)

        start_in(c + 1, tb1, xb1, is1)
        drain_in(tb0, xb0, is0)
        compute(tb0, xb0)
        start_out(c, xb0, os0)

        drain_in(tb1, xb1, is1)

        @pl.when(p < npairs - 1)
        def _():
            drain_out(xb0, os0)
            start_in(c + 2, tb0, xb0, is0)

        compute(tb1, xb1)
        start_out(c + 1, xb1, os1)
        return ()

    lax.fori_loop(0, npairs, pair_body, ())
    drain_out(xb0, os0)
    drain_out(xb1, os1)


def kernel(input_embeddings, emb_table):
    B, S, D = input_embeddings.shape
    mesh = plsc.VectorSubcoreMesh(core_axis_name="c", subcore_axis_name="s")
    k = pl.kernel(
        functools.partial(_body, B, S, D),
        out_type=jax.ShapeDtypeStruct((B * S * D,), input_embeddings.dtype),
        mesh=mesh,
        scratch_types=[
            pltpu.VMEM((R * D,), jnp.float32),
            pltpu.VMEM((R * D,), jnp.float32),
            pltpu.VMEM((B, R * D), jnp.float32),
            pltpu.VMEM((B, R * D), jnp.float32),
            pltpu.SemaphoreType.DMA,
            pltpu.SemaphoreType.DMA,
            pltpu.SemaphoreType.DMA,
            pltpu.SemaphoreType.DMA,
        ],
    )
    out = k(input_embeddings.reshape(-1), emb_table[:S].reshape(-1))
    return out.reshape(B, S, D)


# speedup vs baseline: 6.3392x; 4.4529x over previous
import jax
import jax.numpy as jnp
from jax.experimental import pallas as pl

BS = 2048


def _body(x_ref, t_ref, o_ref):
    o_ref[...] = x_ref[...] + t_ref[...][None, :, :]


def kernel(input_embeddings, emb_table):
    B, S, D = input_embeddings.shape
    ns = S // BS
    return pl.pallas_call(
        _body,
        grid=(ns, B),
        in_specs=[
            pl.BlockSpec((1, BS, D), lambda s, b: (b, s, 0)),
            pl.BlockSpec((BS, D), lambda s, b: (s, 0)),
        ],
        out_specs=pl.BlockSpec((1, BS, D), lambda s, b: (b, s, 0)),
        out_shape=jax.ShapeDtypeStruct((B, S, D), input_embeddings.dtype),
    )(input_embeddings, emb_table[:S])
